# Initial kernel scaffold; baseline (speedup 1.0000x reference)
import jax
import jax.numpy as jnp
from jax.experimental import pallas as pl


def kernel(x, Wd, log_a, scan_idx, mask):
    raise NotImplementedError("write your pallas kernel here")


# speedup vs baseline: 14.7619x; 14.7619x over previous
import functools

import jax
import jax.numpy as jnp
from jax import lax
from jax.experimental import pallas as pl
from jax.experimental.pallas import tpu as pltpu
from jax.experimental.pallas import tpu_sc as plsc

B, C, H, W = 8, 96, 64, 64
HW = H * W
ND = 4
CP = C + 16
K = 128
NCH = HW // K
SLAB = HW + K
PT = 1024
MT = 512


def _mm_body(x_ref, w_ref, o_ref):
    xb = x_ref[0]
    wt = w_ref[0]
    o_ref[0, 0] = lax.dot_general(
        xb, wt, (((0,), (0,)), ((), ())), preferred_element_type=jnp.float32)


def _project(x_flat, wdt):
    return pl.pallas_call(
        _mm_body,
        grid=(B, HW // PT, ND),
        in_specs=[
            pl.BlockSpec((1, C, PT), lambda b, t, d: (b, 0, t)),
            pl.BlockSpec((1, C, C), lambda b, t, d: (d, 0, 0)),
        ],
        out_specs=pl.BlockSpec((1, 1, PT, C), lambda b, t, d: (b, d, t, 0)),
        out_shape=jax.ShapeDtypeStruct((B, ND, HW, C), jnp.float32),
    )(x_flat, wdt)


def _sc_body(xu_hbm, idxg_hbm, idxs_hbm, la_hbm, acc_hbm,
             idx_v, idxg_v, u_buf, ys_buf, la_v, a_v, zb, sem):
    c = lax.axis_index("c")
    s = lax.axis_index("s")
    b = c * 4 + s // 4
    d = s % 4
    w = b * ND + d

    pltpu.sync_copy(la_hbm.at[d], la_v)
    for j in range(C // 16):
        v = la_v[pl.ds(j * 16, 16)]
        a_v[pl.ds(j * 16, 16)] = 1.0 / (1.0 + jnp.exp(-v))

    zvec = jnp.zeros((16,), jnp.float32)

    def zrow(i, _):
        for j in range(CP // 16):
            zb[i, pl.ds(j * 16, 16)] = zvec
        return 0
    lax.fori_loop(0, K, zrow, 0)

    sbase = w * SLAB

    def zchunk(t, _):
        pltpu.sync_copy(zb, acc_hbm.at[pl.ds(sbase + t * K, K)])
        return 0
    lax.fori_loop(0, NCH, zchunk, 0)

    pltpu.sync_copy(idxg_hbm.at[w], idxg_v)
    pltpu.sync_copy(idxs_hbm.at[w], idx_v)

    ovec = jnp.ones((16,), jnp.float32)

    def prow(i, _):
        ys_buf[i, pl.ds(C, 16)] = ovec
        return 0
    lax.fori_loop(0, K, prow, 0)

    a_regs = [a_v[pl.ds(j * 16, 16)] for j in range(C // 16)]

    def chunk(g, h):
        pltpu.async_copy(xu_hbm.at[idxg_v.at[g]], u_buf, sem).wait()

        def srow(l, hh):
            hs = []
            for j in range(C // 16):
                u = u_buf[l, pl.ds(j * 16, 16)]
                nh = a_regs[j] * hh[j] + u
                ys_buf[l, pl.ds(j * 16, 16)] = nh
                hs.append(nh)
            return tuple(hs)
        h = lax.fori_loop(0, K, srow, h)
        pltpu.sync_copy(ys_buf, acc_hbm.at[idx_v.at[g]])
        return h

    h0 = tuple(jnp.zeros((16,), jnp.float32) for _ in range(C // 16))
    lax.fori_loop(0, NCH, chunk, h0)


@functools.partial(
    pl.kernel,
    out_type=jax.ShapeDtypeStruct((B * ND * SLAB, CP), jnp.float32),
    mesh=plsc.VectorSubcoreMesh(core_axis_name="c", subcore_axis_name="s"),
    scratch_types=[
        pltpu.VMEM((NCH, K), jnp.int32),
        pltpu.VMEM((NCH, K), jnp.int32),
        pltpu.VMEM((K, C), jnp.float32),
        pltpu.VMEM((K, CP), jnp.float32),
        pltpu.VMEM((C,), jnp.float32),
        pltpu.VMEM((C,), jnp.float32),
        pltpu.VMEM((K, CP), jnp.float32),
        pltpu.SemaphoreType.DMA,
    ],
    compiler_params=pltpu.CompilerParams(use_tc_tiling_on_sc=False),
)
def _sc_ragged(xu_flat, idxg, idxs, log_a, acc, *scratch):
    _sc_body(xu_flat, idxg, idxs, log_a, acc, *scratch)


def _merge_body(acc_ref, o_ref):
    blk = acc_ref[0]
    tot = blk[0] + blk[1] + blk[2] + blk[3]
    vals = tot[:, :C]
    cnt = tot[:, C:C + 1]
    o_ref[0] = vals / (cnt + 1e-6)


def _merge(acc4):
    return pl.pallas_call(
        _merge_body,
        grid=(B, HW // MT),
        in_specs=[pl.BlockSpec((1, ND, MT, CP), lambda b, t: (b, 0, t, 0))],
        out_specs=pl.BlockSpec((1, MT, C), lambda b, t: (b, t, 0)),
        out_shape=jax.ShapeDtypeStruct((B, HW, C), jnp.float32),
    )(acc4)


def kernel(x, Wd, log_a, scan_idx, mask):
    x_flat = x.reshape(B, C, HW)
    wdt = jnp.transpose(Wd, (0, 2, 1))
    xu = _project(x_flat, wdt)
    xu_flat = xu.reshape(B * ND * HW, C)

    woff = (jnp.arange(B, dtype=jnp.int32) * ND)[:, None] \
        + jnp.arange(ND, dtype=jnp.int32)[None, :]
    idxg = jnp.minimum(scan_idx, HW - 1) + (woff * HW)[:, :, None]
    idxs = scan_idx + (woff * SLAB)[:, :, None]
    del mask

    acc = _sc_ragged(
        xu_flat,
        idxg.reshape(B * ND, NCH, K),
        idxs.reshape(B * ND, NCH, K),
        log_a,
    )
    acc4 = acc.reshape(B, ND, SLAB, CP)
    out = _merge(acc4)
    return jnp.transpose(out.reshape(B, H, W, C), (0, 3, 1, 2))


# speedup vs baseline: 16.4701x; 1.1157x over previous
import functools

import jax
import jax.numpy as jnp
from jax import lax
from jax.experimental import pallas as pl
from jax.experimental.pallas import tpu as pltpu
from jax.experimental.pallas import tpu_sc as plsc

B, C, H, W = 8, 96, 64, 64
HW = H * W
ND = 4
CP = C + 16
K = 128
NCH = HW // K
SLAB = HW + K
QR = HW // 4
MR = K // ND
PT = 1024
NJ = C // 16


def _mm_body(x_ref, w_ref, o_ref):
    xb = x_ref[0]
    wt = w_ref[0]
    o_ref[0, 0] = lax.dot_general(
        xb, wt, (((0,), (0,)), ((), ())), preferred_element_type=jnp.float32)


def _project(x_flat, wdt):
    return pl.pallas_call(
        _mm_body,
        grid=(B, HW // PT, ND),
        in_specs=[
            pl.BlockSpec((1, C, PT), lambda b, t, d: (b, 0, t)),
            pl.BlockSpec((1, C, C), lambda b, t, d: (d, 0, 0)),
        ],
        out_specs=pl.BlockSpec((1, 1, PT, C), lambda b, t, d: (b, d, t, 0)),
        out_shape=jax.ShapeDtypeStruct((B, ND, HW, C), jnp.float32),
        compiler_params=pltpu.CompilerParams(fuse_transposed_lhs_in_matmul=True),
    )(x_flat, wdt)


def _sc_body(xu_hbm, idxg_hbm, idxs_hbm, midx_hbm, la_hbm,
             acc_hbm, out_hbm,
             idx_v, idxg_v, u_buf, ys_buf, la_v, a_v, zb,
             sem_g, sem_s, sem_z, sem_o):
    c = lax.axis_index("c")
    s = lax.axis_index("s")
    b = c * 4 + s // 4
    d = s % 4
    q = s % 4
    w = b * ND + d
    wid = c * 16 + s

    pltpu.sync_copy(idxg_hbm.at[w], idxg_v)
    pltpu.sync_copy(idxs_hbm.at[w], idx_v)
    pltpu.sync_copy(la_hbm.at[d], la_v)

    zvec = jnp.zeros((16,), jnp.float32)

    def zrow(i, _):
        for j in range(CP // 16):
            zb[i, pl.ds(j * 16, 16)] = zvec
        return 0
    lax.fori_loop(0, K, zrow, 0)

    sbase = w * SLAB

    def zfire(t, _):
        pltpu.async_copy(zb, acc_hbm.at[pl.ds(sbase + t * K, K)], sem_z)
        return 0
    lax.fori_loop(0, NCH, zfire, 0)

    for j in range(NJ):
        v = la_v[pl.ds(j * 16, 16)]
        a_v[pl.ds(j * 16, 16)] = 1.0 / (1.0 + jnp.exp(-v))

    ovec = jnp.ones((16,), jnp.float32)

    def prow(i, _):
        ys_buf[0, i, pl.ds(C, 16)] = ovec
        ys_buf[1, i, pl.ds(C, 16)] = ovec
        return 0
    lax.fori_loop(0, K, prow, 0)

    def zdrain(t, _):
        pltpu.make_async_copy(zb, acc_hbm.at[pl.ds(sbase, K)], sem_z).wait()
        return 0
    lax.fori_loop(0, NCH, zdrain, 0)

    a_regs = [a_v[pl.ds(j * 16, 16)] for j in range(NJ)]

    pltpu.async_copy(xu_hbm.at[idxg_v.at[0]], u_buf.at[0], sem_g)

    NG2 = NCH // 2

    def chunk2(o, h):
        for i in range(2):
            g = o * 2 + i
            pltpu.make_async_copy(
                xu_hbm.at[idxg_v.at[g]], u_buf.at[i], sem_g).wait()
            if i == 0:
                pltpu.async_copy(
                    xu_hbm.at[idxg_v.at[g + 1]], u_buf.at[1], sem_g)
            else:
                @pl.when(o < NG2 - 1)
                def _():
                    pltpu.async_copy(
                        xu_hbm.at[idxg_v.at[g + 1]], u_buf.at[0], sem_g)

            @pl.when(o >= 1)
            def _():
                pltpu.make_async_copy(
                    ys_buf.at[i], acc_hbm.at[idx_v.at[g - 2]], sem_s).wait()

            def srow(l, hh):
                hs = []
                for j in range(NJ):
                    u = u_buf[i, l, pl.ds(j * 16, 16)]
                    nh = a_regs[j] * hh[j] + u
                    ys_buf[i, l, pl.ds(j * 16, 16)] = nh
                    hs.append(nh)
                return tuple(hs)
            h = lax.fori_loop(0, K, srow, h)

            pltpu.async_copy(ys_buf.at[i], acc_hbm.at[idx_v.at[g]], sem_s)
        return h

    h0 = tuple(jnp.zeros((16,), jnp.float32) for _ in range(NJ))
    lax.fori_loop(0, NG2, chunk2, h0)

    for i in range(2):
        pltpu.make_async_copy(
            ys_buf.at[i], acc_hbm.at[idx_v.at[NCH - 2 + i]], sem_s).wait()

    plsc.subcore_barrier()

    pltpu.sync_copy(midx_hbm.at[wid], idxg_v)
    obase = q * QR
    NM = QR // MR
    NM2 = NM // 2

    pltpu.async_copy(acc_hbm.at[idxg_v.at[0]], ys_buf.at[0], sem_g)

    def merge2(o, _):
        for i in range(2):
            m = o * 2 + i
            pltpu.make_async_copy(
                acc_hbm.at[idxg_v.at[m]], ys_buf.at[i], sem_g).wait()
            if i == 0:
                pltpu.async_copy(
                    acc_hbm.at[idxg_v.at[m + 1]], ys_buf.at[1], sem_g)
            else:
                @pl.when(o < NM2 - 1)
                def _():
                    pltpu.async_copy(
                        acc_hbm.at[idxg_v.at[m + 1]], ys_buf.at[0], sem_g)

            @pl.when(o >= 1)
            def _():
                pltpu.make_async_copy(
                    u_buf.at[i].at[pl.ds(0, MR)],
                    out_hbm.at[b, pl.ds(obase + (m - 2) * MR, MR)],
                    sem_o).wait()

            def mrow(r, _2):
                cnt = (ys_buf[i, r, pl.ds(C, 16)]
                       + ys_buf[i, MR + r, pl.ds(C, 16)]
                       + ys_buf[i, 2 * MR + r, pl.ds(C, 16)]
                       + ys_buf[i, 3 * MR + r, pl.ds(C, 16)])
                inv = 1.0 / (cnt + 1e-6)
                for j in range(NJ):
                    tot = (ys_buf[i, r, pl.ds(j * 16, 16)]
                           + ys_buf[i, MR + r, pl.ds(j * 16, 16)]
                           + ys_buf[i, 2 * MR + r, pl.ds(j * 16, 16)]
                           + ys_buf[i, 3 * MR + r, pl.ds(j * 16, 16)])
                    u_buf[i, r, pl.ds(j * 16, 16)] = tot * inv
                return 0
            lax.fori_loop(0, MR, mrow, 0)

            pltpu.async_copy(
                u_buf.at[i].at[pl.ds(0, MR)],
                out_hbm.at[b, pl.ds(obase + m * MR, MR)],
                sem_o)
        return 0

    lax.fori_loop(0, NM2, merge2, 0)

    for i in range(2):
        pltpu.make_async_copy(
            u_buf.at[i].at[pl.ds(0, MR)],
            out_hbm.at[b, pl.ds(obase + (NM - 2 + i) * MR, MR)],
            sem_o).wait()


@functools.partial(
    pl.kernel,
    out_type=(
        jax.ShapeDtypeStruct((B * ND * SLAB, CP), jnp.float32),
        jax.ShapeDtypeStruct((B, HW, C), jnp.float32),
    ),
    mesh=plsc.VectorSubcoreMesh(core_axis_name="c", subcore_axis_name="s"),
    scratch_types=[
        pltpu.VMEM((NCH, K), jnp.int32),
        pltpu.VMEM((NCH, K), jnp.int32),
        pltpu.VMEM((2, K, C), jnp.float32),
        pltpu.VMEM((2, K, CP), jnp.float32),
        pltpu.VMEM((C,), jnp.float32),
        pltpu.VMEM((C,), jnp.float32),
        pltpu.VMEM((K, CP), jnp.float32),
        pltpu.SemaphoreType.DMA,
        pltpu.SemaphoreType.DMA,
        pltpu.SemaphoreType.DMA,
        pltpu.SemaphoreType.DMA,
    ],
    compiler_params=pltpu.CompilerParams(use_tc_tiling_on_sc=False),
)
def _sc_ragged(xu_flat, idxg, idxs, midx, log_a, acc, out, *scratch):
    _sc_body(xu_flat, idxg, idxs, midx, log_a, acc, out, *scratch)


def _merge_index_table():
    wid = jnp.arange(32, dtype=jnp.int32)[:, None, None]
    b = (wid // 16) * 4 + (wid % 16) // 4
    q = wid % 4
    t = jnp.arange(NM_T := QR // MR, dtype=jnp.int32)[None, :, None]
    p = jnp.arange(K, dtype=jnp.int32)[None, None, :]
    dd = p // MR
    r = p % MR
    return (b * ND + dd) * SLAB + q * QR + t * MR + r


def kernel(x, Wd, log_a, scan_idx, mask):
    x_flat = x.reshape(B, C, HW)
    wdt = jnp.transpose(Wd, (0, 2, 1))
    xu = _project(x_flat, wdt)
    xu_flat = xu.reshape(B * ND * HW, C)

    woff = (jnp.arange(B, dtype=jnp.int32) * ND)[:, None] \
        + jnp.arange(ND, dtype=jnp.int32)[None, :]
    idxg = jnp.minimum(scan_idx, HW - 1) + (woff * HW)[:, :, None]
    idxs = scan_idx + (woff * SLAB)[:, :, None]
    midx = _merge_index_table()
    del mask

    _, out = _sc_ragged(
        xu_flat,
        idxg.reshape(B * ND, NCH, K),
        idxs.reshape(B * ND, NCH, K),
        midx,
        log_a,
    )
    return jnp.transpose(out.reshape(B, H, W, C), (0, 3, 1, 2))


# speedup vs baseline: 18.9769x; 1.1522x over previous
import functools

import jax
import jax.numpy as jnp
from jax import lax
from jax.experimental import pallas as pl
from jax.experimental.pallas import tpu as pltpu
from jax.experimental.pallas import tpu_sc as plsc

B, C, H, W = 8, 96, 64, 64
HW = H * W
ND = 4
CP = C + 16
K = 128
NCH = HW // K
SLAB = HW + K
QR = HW // 4
MR = K // ND
PT = 2048
NJ = C // 16
NB = 4


def _mm_body(x_ref, w_ref, o_ref):
    xb = x_ref[0]
    wt = w_ref[0]
    o_ref[0, 0] = lax.dot_general(
        xb, wt, (((0,), (0,)), ((), ())), preferred_element_type=jnp.float32)


def _project(x_flat, wdt):
    return pl.pallas_call(
        _mm_body,
        grid=(B, HW // PT, ND),
        in_specs=[
            pl.BlockSpec((1, C, PT), lambda b, t, d: (b, 0, t)),
            pl.BlockSpec((1, C, C), lambda b, t, d: (d, 0, 0)),
        ],
        out_specs=pl.BlockSpec((1, 1, PT, C), lambda b, t, d: (b, d, t, 0)),
        out_shape=jax.ShapeDtypeStruct((B, ND, HW, C), jnp.float32),
        compiler_params=pltpu.CompilerParams(fuse_transposed_lhs_in_matmul=True),
    )(x_flat, wdt)


def _sc_body(xu_hbm, idxg_hbm, idxs_hbm, midx_hbm, la_hbm,
             acc_hbm, out_hbm,
             idx_v, idxg_v, u_buf, ys_buf, la_v, a_v,
             sem_g, sem_s, sem_z, sem_o):
    c = lax.axis_index("c")
    s = lax.axis_index("s")
    b = c * 4 + s // 4
    d = s % 4
    q = s % 4
    w = b * ND + d
    wid = c * 16 + s

    pltpu.sync_copy(idxg_hbm.at[w], idxg_v)
    pltpu.sync_copy(idxs_hbm.at[w], idx_v)
    pltpu.sync_copy(la_hbm.at[d], la_v)

    zvec = jnp.zeros((16,), jnp.float32)

    def zrow(i, _):
        for j in range(CP // 16):
            ys_buf[0, i, pl.ds(j * 16, 16)] = zvec
        return 0
    lax.fori_loop(0, K, zrow, 0)

    sbase = w * SLAB
    zb = ys_buf.at[0]

    def zfire(t, _):
        pltpu.async_copy(zb, acc_hbm.at[pl.ds(sbase + t * K, K)], sem_z)
        return 0
    lax.fori_loop(0, NCH, zfire, 0)

    for j in range(NJ):
        v = la_v[pl.ds(j * 16, 16)]
        a_v[pl.ds(j * 16, 16)] = 1.0 / (1.0 + jnp.exp(-v))

    ovec = jnp.ones((16,), jnp.float32)

    def prow(i, _):
        for n in range(1, NB):
            ys_buf[n, i, pl.ds(C, 16)] = ovec
        return 0
    lax.fori_loop(0, K, prow, 0)

    def zdrain(t, _):
        pltpu.make_async_copy(zb, acc_hbm.at[pl.ds(sbase, K)], sem_z).wait()
        return 0
    lax.fori_loop(0, NCH, zdrain, 0)

    def prow0(i, _):
        ys_buf[0, i, pl.ds(C, 16)] = ovec
        return 0
    lax.fori_loop(0, K, prow0, 0)

    a_regs = [a_v[pl.ds(j * 16, 16)] for j in range(NJ)]

    for n in range(NB - 1):
        pltpu.async_copy(xu_hbm.at[idxg_v.at[n]], u_buf.at[n], sem_g)

    NGO = NCH // NB

    def chunk4(o, h):
        for i in range(NB):
            g = o * NB + i
            pltpu.make_async_copy(
                xu_hbm.at[idxg_v.at[g]], u_buf.at[i], sem_g).wait()
            if i == 0:
                pltpu.async_copy(
                    xu_hbm.at[idxg_v.at[g + NB - 1]],
                    u_buf.at[NB - 1], sem_g)
            else:
                @pl.when(g + NB - 1 < NCH)
                def _():
                    pltpu.async_copy(
                        xu_hbm.at[idxg_v.at[g + NB - 1]],
                        u_buf.at[i - 1], sem_g)

            @pl.when(o >= 1)
            def _():
                pltpu.make_async_copy(
                    ys_buf.at[i], acc_hbm.at[idx_v.at[g - NB]], sem_s).wait()

            def srow(l, hh):
                hs = []
                for j in range(NJ):
                    u = u_buf[i, l, pl.ds(j * 16, 16)]
                    nh = a_regs[j] * hh[j] + u
                    ys_buf[i, l, pl.ds(j * 16, 16)] = nh
                    hs.append(nh)
                return tuple(hs)
            h = lax.fori_loop(0, K, srow, h)

            pltpu.async_copy(ys_buf.at[i], acc_hbm.at[idx_v.at[g]], sem_s)
        return h

    h0 = tuple(jnp.zeros((16,), jnp.float32) for _ in range(NJ))
    lax.fori_loop(0, NGO, chunk4, h0)

    for i in range(NB):
        pltpu.make_async_copy(
            ys_buf.at[i], acc_hbm.at[idx_v.at[NCH - NB + i]], sem_s).wait()

    plsc.subcore_barrier()

    pltpu.sync_copy(midx_hbm.at[wid], idxg_v)
    obase = q * QR
    NM = QR // MR
    NMO = NM // NB

    for n in range(NB - 1):
        pltpu.async_copy(acc_hbm.at[idxg_v.at[n]], ys_buf.at[n], sem_g)

    def merge4(o, _):
        for i in range(NB):
            m = o * NB + i
            pltpu.make_async_copy(
                acc_hbm.at[idxg_v.at[m]], ys_buf.at[i], sem_g).wait()
            if i == 0:
                pltpu.async_copy(
                    acc_hbm.at[idxg_v.at[m + NB - 1]],
                    ys_buf.at[NB - 1], sem_g)
            else:
                @pl.when(m + NB - 1 < NM)
                def _():
                    pltpu.async_copy(
                        acc_hbm.at[idxg_v.at[m + NB - 1]],
                        ys_buf.at[i - 1], sem_g)

            @pl.when(o >= 1)
            def _():
                pltpu.make_async_copy(
                    u_buf.at[i].at[pl.ds(0, MR)],
                    out_hbm.at[b, pl.ds(obase + (m - NB) * MR, MR)],
                    sem_o).wait()

            def mrow(r, _2):
                cnt = (ys_buf[i, r, pl.ds(C, 16)]
                       + ys_buf[i, MR + r, pl.ds(C, 16)]
                       + ys_buf[i, 2 * MR + r, pl.ds(C, 16)]
                       + ys_buf[i, 3 * MR + r, pl.ds(C, 16)])
                inv = 1.0 / (cnt + 1e-6)
                for j in range(NJ):
                    tot = (ys_buf[i, r, pl.ds(j * 16, 16)]
                           + ys_buf[i, MR + r, pl.ds(j * 16, 16)]
                           + ys_buf[i, 2 * MR + r, pl.ds(j * 16, 16)]
                           + ys_buf[i, 3 * MR + r, pl.ds(j * 16, 16)])
                    u_buf[i, r, pl.ds(j * 16, 16)] = tot * inv
                return 0
            lax.fori_loop(0, MR, mrow, 0)

            pltpu.async_copy(
                u_buf.at[i].at[pl.ds(0, MR)],
                out_hbm.at[b, pl.ds(obase + m * MR, MR)],
                sem_o)
        return 0

    lax.fori_loop(0, NMO, merge4, 0)

    for i in range(NB):
        pltpu.make_async_copy(
            u_buf.at[i].at[pl.ds(0, MR)],
            out_hbm.at[b, pl.ds(obase + (NM - NB + i) * MR, MR)],
            sem_o).wait()


@functools.partial(
    pl.kernel,
    out_type=(
        jax.ShapeDtypeStruct((B * ND * SLAB, CP), jnp.float32),
        jax.ShapeDtypeStruct((B, HW, C), jnp.float32),
    ),
    mesh=plsc.VectorSubcoreMesh(core_axis_name="c", subcore_axis_name="s"),
    scratch_types=[
        pltpu.VMEM((NCH, K), jnp.int32),
        pltpu.VMEM((NCH, K), jnp.int32),
        pltpu.VMEM((NB, K, C), jnp.float32),
        pltpu.VMEM((NB, K, CP), jnp.float32),
        pltpu.VMEM((C,), jnp.float32),
        pltpu.VMEM((C,), jnp.float32),
        pltpu.SemaphoreType.DMA,
        pltpu.SemaphoreType.DMA,
        pltpu.SemaphoreType.DMA,
        pltpu.SemaphoreType.DMA,
    ],
    compiler_params=pltpu.CompilerParams(use_tc_tiling_on_sc=False),
)
def _sc_ragged(xu_flat, idxg, idxs, midx, log_a, acc, out, *scratch):
    _sc_body(xu_flat, idxg, idxs, midx, log_a, acc, out, *scratch)


def _merge_index_table():
    wid = jnp.arange(32, dtype=jnp.int32)[:, None, None]
    b = (wid // 16) * 4 + (wid % 16) // 4
    q = wid % 4
    t = jnp.arange(QR // MR, dtype=jnp.int32)[None, :, None]
    p = jnp.arange(K, dtype=jnp.int32)[None, None, :]
    dd = p // MR
    r = p % MR
    return (b * ND + dd) * SLAB + q * QR + t * MR + r


def kernel(x, Wd, log_a, scan_idx, mask):
    x_flat = x.reshape(B, C, HW)
    wdt = jnp.transpose(Wd, (0, 2, 1))
    xu = _project(x_flat, wdt)
    xu_flat = xu.reshape(B * ND * HW, C)

    woff = (jnp.arange(B, dtype=jnp.int32) * ND)[:, None] \
        + jnp.arange(ND, dtype=jnp.int32)[None, :]
    idxg = jnp.minimum(scan_idx, HW - 1) + (woff * HW)[:, :, None]
    idxs = scan_idx + (woff * SLAB)[:, :, None]
    midx = _merge_index_table()
    del mask

    _, out = _sc_ragged(
        xu_flat,
        idxg.reshape(B * ND, NCH, K),
        idxs.reshape(B * ND, NCH, K),
        midx,
        log_a,
    )
    return jnp.transpose(out.reshape(B, H, W, C), (0, 3, 1, 2))


# speedup vs baseline: 18.9852x; 1.0004x over previous
import functools

import jax
import jax.numpy as jnp
from jax import lax
from jax.experimental import pallas as pl
from jax.experimental.pallas import tpu as pltpu
from jax.experimental.pallas import tpu_sc as plsc

B, C, H, W = 8, 96, 64, 64
HW = H * W
ND = 4
CP = C + 16
K = 128
NCH = HW // K
SLAB = HW + K
QR = HW // 4
MR = K // ND
PT = 2048
NJ = C // 16
NB = 4


def _mm_body(x_ref, w_ref, o_ref):
    xb = x_ref[0]
    wt = w_ref[0]
    o_ref[0, 0] = lax.dot_general(
        xb, wt, (((0,), (0,)), ((), ())), preferred_element_type=jnp.float32)


def _project(x_flat, wdt):
    return pl.pallas_call(
        _mm_body,
        grid=(B, HW // PT, ND),
        in_specs=[
            pl.BlockSpec((1, C, PT), lambda b, t, d: (b, 0, t)),
            pl.BlockSpec((1, C, C), lambda b, t, d: (d, 0, 0)),
        ],
        out_specs=pl.BlockSpec((1, 1, PT, C), lambda b, t, d: (b, d, t, 0)),
        out_shape=jax.ShapeDtypeStruct((B, ND, HW, C), jnp.float32),
        compiler_params=pltpu.CompilerParams(fuse_transposed_lhs_in_matmul=True),
    )(x_flat, wdt)


def _sc_body(xu_hbm, idxg_hbm, idxs_hbm, midx_hbm, la_hbm,
             acc_hbm, out_hbm,
             idx_v, idxg_v, u_buf, ys_buf, la_v, a_v,
             sem_g, sem_s, sem_z, sem_o):
    c = lax.axis_index("c")
    s = lax.axis_index("s")
    b = c * 4 + s // 4
    d = s % 4
    q = s % 4
    w = b * ND + d
    wid = c * 16 + s

    with jax.named_scope("sc_setup_zero"):
        pltpu.sync_copy(idxg_hbm.at[w], idxg_v)
        pltpu.sync_copy(idxs_hbm.at[w], idx_v)
        pltpu.sync_copy(la_hbm.at[d], la_v)

        zvec = jnp.zeros((16,), jnp.float32)

        def zrow(i, _):
            for j in range(CP // 16):
                ys_buf[0, i, pl.ds(j * 16, 16)] = zvec
            return 0
        lax.fori_loop(0, K, zrow, 0)

        sbase = w * SLAB
        zb = ys_buf.at[0]

        def zfire(t, _):
            pltpu.async_copy(zb, acc_hbm.at[pl.ds(sbase + t * K, K)], sem_z)
            return 0
        lax.fori_loop(0, NCH, zfire, 0)

        for j in range(NJ):
            v = la_v[pl.ds(j * 16, 16)]
            a_v[pl.ds(j * 16, 16)] = 1.0 / (1.0 + jnp.exp(-v))

        ovec = jnp.ones((16,), jnp.float32)

        def prow(i, _):
            for n in range(1, NB):
                ys_buf[n, i, pl.ds(C, 16)] = ovec
            return 0
        lax.fori_loop(0, K, prow, 0)

        def zdrain(t, _):
            pltpu.make_async_copy(zb, acc_hbm.at[pl.ds(sbase, K)], sem_z).wait()
            return 0
        lax.fori_loop(0, NCH, zdrain, 0)

        def prow0(i, _):
            ys_buf[0, i, pl.ds(C, 16)] = ovec
            return 0
        lax.fori_loop(0, K, prow0, 0)

    a_regs = [a_v[pl.ds(j * 16, 16)] for j in range(NJ)]

    for n in range(NB - 1):
        pltpu.async_copy(xu_hbm.at[idxg_v.at[n]], u_buf.at[n], sem_g)

    NGO = NCH // NB

    def chunk4(o, h):
        for i in range(NB):
            g = o * NB + i
            pltpu.make_async_copy(
                xu_hbm.at[idxg_v.at[g]], u_buf.at[i], sem_g).wait()
            if i == 0:
                pltpu.async_copy(
                    xu_hbm.at[idxg_v.at[g + NB - 1]],
                    u_buf.at[NB - 1], sem_g)
            else:
                @pl.when(g + NB - 1 < NCH)
                def _():
                    pltpu.async_copy(
                        xu_hbm.at[idxg_v.at[g + NB - 1]],
                        u_buf.at[i - 1], sem_g)

            @pl.when(o >= 1)
            def _():
                pltpu.make_async_copy(
                    ys_buf.at[i], acc_hbm.at[idx_v.at[g - NB]], sem_s).wait()

            def srow(l, hh):
                hs = []
                for j in range(NJ):
                    u = u_buf[i, l, pl.ds(j * 16, 16)]
                    nh = a_regs[j] * hh[j] + u
                    ys_buf[i, l, pl.ds(j * 16, 16)] = nh
                    hs.append(nh)
                return tuple(hs)
            h = lax.fori_loop(0, K, srow, h)

            pltpu.async_copy(ys_buf.at[i], acc_hbm.at[idx_v.at[g]], sem_s)
        return h

    h0 = tuple(jnp.zeros((16,), jnp.float32) for _ in range(NJ))
    with jax.named_scope("sc_mainloop"):
        lax.fori_loop(0, NGO, chunk4, h0)

        for i in range(NB):
            pltpu.make_async_copy(
                ys_buf.at[i], acc_hbm.at[idx_v.at[NCH - NB + i]], sem_s).wait()

    with jax.named_scope("sc_barrier"):
        plsc.subcore_barrier()

    jax.named_scope("sc_merge").__enter__()
    pltpu.sync_copy(midx_hbm.at[wid], idxg_v)
    obase = q * QR
    NM = QR // MR
    NMO = NM // NB

    for n in range(NB - 1):
        pltpu.async_copy(acc_hbm.at[idxg_v.at[n]], ys_buf.at[n], sem_g)

    def merge4(o, _):
        for i in range(NB):
            m = o * NB + i
            pltpu.make_async_copy(
                acc_hbm.at[idxg_v.at[m]], ys_buf.at[i], sem_g).wait()
            if i == 0:
                pltpu.async_copy(
                    acc_hbm.at[idxg_v.at[m + NB - 1]],
                    ys_buf.at[NB - 1], sem_g)
            else:
                @pl.when(m + NB - 1 < NM)
                def _():
                    pltpu.async_copy(
                        acc_hbm.at[idxg_v.at[m + NB - 1]],
                        ys_buf.at[i - 1], sem_g)

            @pl.when(o >= 1)
            def _():
                pltpu.make_async_copy(
                    u_buf.at[i].at[pl.ds(0, MR)],
                    out_hbm.at[b, pl.ds(obase + (m - NB) * MR, MR)],
                    sem_o).wait()

            def mrow(r, _2):
                cnt = (ys_buf[i, r, pl.ds(C, 16)]
                       + ys_buf[i, MR + r, pl.ds(C, 16)]
                       + ys_buf[i, 2 * MR + r, pl.ds(C, 16)]
                       + ys_buf[i, 3 * MR + r, pl.ds(C, 16)])
                inv = 1.0 / (cnt + 1e-6)
                for j in range(NJ):
                    tot = (ys_buf[i, r, pl.ds(j * 16, 16)]
                           + ys_buf[i, MR + r, pl.ds(j * 16, 16)]
                           + ys_buf[i, 2 * MR + r, pl.ds(j * 16, 16)]
                           + ys_buf[i, 3 * MR + r, pl.ds(j * 16, 16)])
                    u_buf[i, r, pl.ds(j * 16, 16)] = tot * inv
                return 0
            lax.fori_loop(0, MR, mrow, 0)

            pltpu.async_copy(
                u_buf.at[i].at[pl.ds(0, MR)],
                out_hbm.at[b, pl.ds(obase + m * MR, MR)],
                sem_o)
        return 0

    lax.fori_loop(0, NMO, merge4, 0)

    for i in range(NB):
        pltpu.make_async_copy(
            u_buf.at[i].at[pl.ds(0, MR)],
            out_hbm.at[b, pl.ds(obase + (NM - NB + i) * MR, MR)],
            sem_o).wait()


@functools.partial(
    pl.kernel,
    out_type=(
        jax.ShapeDtypeStruct((B * ND * SLAB, CP), jnp.float32),
        jax.ShapeDtypeStruct((B, HW, C), jnp.float32),
    ),
    mesh=plsc.VectorSubcoreMesh(core_axis_name="c", subcore_axis_name="s"),
    scratch_types=[
        pltpu.VMEM((NCH, K), jnp.int32),
        pltpu.VMEM((NCH, K), jnp.int32),
        pltpu.VMEM((NB, K, C), jnp.float32),
        pltpu.VMEM((NB, K, CP), jnp.float32),
        pltpu.VMEM((C,), jnp.float32),
        pltpu.VMEM((C,), jnp.float32),
        pltpu.SemaphoreType.DMA,
        pltpu.SemaphoreType.DMA,
        pltpu.SemaphoreType.DMA,
        pltpu.SemaphoreType.DMA,
    ],
    compiler_params=pltpu.CompilerParams(use_tc_tiling_on_sc=False),
)
def _sc_ragged(xu_flat, idxg, idxs, midx, log_a, acc, out, *scratch):
    _sc_body(xu_flat, idxg, idxs, midx, log_a, acc, out, *scratch)


def _merge_index_table():
    wid = jnp.arange(32, dtype=jnp.int32)[:, None, None]
    b = (wid // 16) * 4 + (wid % 16) // 4
    q = wid % 4
    t = jnp.arange(QR // MR, dtype=jnp.int32)[None, :, None]
    p = jnp.arange(K, dtype=jnp.int32)[None, None, :]
    dd = p // MR
    r = p % MR
    return (b * ND + dd) * SLAB + q * QR + t * MR + r


def kernel(x, Wd, log_a, scan_idx, mask):
    x_flat = x.reshape(B, C, HW)
    wdt = jnp.transpose(Wd, (0, 2, 1))
    xu = _project(x_flat, wdt)
    xu_flat = xu.reshape(B * ND * HW, C)

    woff = (jnp.arange(B, dtype=jnp.int32) * ND)[:, None] \
        + jnp.arange(ND, dtype=jnp.int32)[None, :]
    idxg = jnp.minimum(scan_idx, HW - 1) + (woff * HW)[:, :, None]
    idxs = scan_idx + (woff * SLAB)[:, :, None]
    midx = _merge_index_table()
    del mask

    _, out = _sc_ragged(
        xu_flat,
        idxg.reshape(B * ND, NCH, K),
        idxs.reshape(B * ND, NCH, K),
        midx,
        log_a,
    )
    return jnp.transpose(out.reshape(B, H, W, C), (0, 3, 1, 2))
